# baseline (speedup 1.0000x reference)
import functools

import numpy as np

import jax
import jax.numpy as jnp
from jax import lax
from jax.experimental import pallas as pl
from jax.experimental.pallas import tpu as pltpu
from jax.experimental.pallas import tpu_sc as plsc

N = 10000
E = 320000
F = 128

NUM_CORES = 2
NUM_SUBCORES = 16
NUM_TILES = NUM_CORES * NUM_SUBCORES

EDGES_PER_TILE = E // NUM_TILES
CHUNK = 80
NCHUNKS = EDGES_PER_TILE // CHUNK
NPAD = 10240
ROWS_PER_TILE = NPAD // NUM_SUBCORES

M_BLK = 1000

NBUF = 2
IDX_BITS = 14
IDX_MASK = (1 << IDX_BITS) - 1

_PERM = np.concatenate(
    [np.concatenate([32 * j + np.arange(0, 32, 2), 32 * j + np.arange(1, 32, 2)])
     for j in range(F // 32)])


def _mm1_kernel(x_ref, w_ref, b_ref, o_ref):
    o_ref[...] = lax.dot_general(
        x_ref[...], w_ref[...], (((1,), (1,)), ((), ())),
        preferred_element_type=jnp.float32) + b_ref[...]


def _mm2_kernel(a_ref, w_ref, b_ref, o_ref):
    a = a_ref[0] + a_ref[1]
    o_ref[...] = lax.dot_general(
        a, w_ref[...], (((1,), (1,)), ((), ())),
        preferred_element_type=jnp.float32) + b_ref[...]


def _sc_body(hp_hbm, packed_hbm, ew_hbm, out_hbm,
             packed_all, w_all, sidx_b, didx_b, rows_g, fbufs, acc,
             gsems, ssems):
    c = lax.axis_index("c")
    s = lax.axis_index("s")
    wid = c * NUM_SUBCORES + s

    def _fire_gather(ck, b):
        for g in range(CHUNK // 16):
            pv = packed_all[pl.ds(ck * CHUNK + g * 16, 16)]
            sidx_b[b, pl.ds(g * 16, 16)] = pv & IDX_MASK
            didx_b[b, pl.ds(g * 16, 16)] = lax.shift_right_logical(pv, IDX_BITS)
        pltpu.async_copy(hp_hbm.at[sidx_b.at[b]], rows_g.at[b], gsems.at[b])

    def _wait_gather(b):
        pltpu.make_async_copy(hp_hbm.at[sidx_b.at[b]], rows_g.at[b],
                              gsems.at[b]).wait()

    def _widen(v32):
        lo = lax.bitcast_convert_type(lax.shift_left(v32, 16), jnp.float32)
        hi = lax.bitcast_convert_type(v32 & jnp.int32(-65536), jnp.float32)
        return lo, hi

    def _scale_edge(rows_b, fb, k, wk):
        for j in range(F // 32):
            v32 = rows_b[k, pl.ds(j * 16, 16)]
            ua, ub = _widen(v32)
            fb[k, pl.ds((2 * j) * 16, 16)] = ua * wk
            fb[k, pl.ds((2 * j + 1) * 16, 16)] = ub * wk

    def _scale(ck, b):
        rows_b = rows_g.at[b]
        fb = fbufs.at[b]
        wbase = ck * (CHUNK // 2)
        for t in range(3):
            wv32 = w_all[pl.ds(wbase + t * 16, 16)]
            wlo, whi = _widen(wv32)
            for m in range(16 if t < 2 else 8):
                _scale_edge(rows_b, fb, 32 * t + 2 * m, wlo[m])
                _scale_edge(rows_b, fb, 32 * t + 2 * m + 1, whi[m])

    pltpu.sync_copy(packed_hbm.at[pl.ds(wid * EDGES_PER_TILE, EDGES_PER_TILE)],
                    packed_all)
    pltpu.sync_copy(ew_hbm.at[pl.ds(wid * (EDGES_PER_TILE // 2),
                                    EDGES_PER_TILE // 2)],
                    w_all.at[pl.ds(0, EDGES_PER_TILE // 2)])

    def _zero_body(i, _):
        z = jnp.zeros((16,), jnp.float32)
        for j in range(F // 16):
            fbufs[0, i, pl.ds(j * 16, 16)] = z
        return 0
    lax.fori_loop(0, CHUNK, _zero_body, 0)
    for t in range(ROWS_PER_TILE // CHUNK):
        pltpu.sync_copy(fbufs.at[0],
                        acc.at[pl.ds(s * ROWS_PER_TILE + t * CHUNK, CHUNK)])
    plsc.subcore_barrier()

    for b in range(NBUF):
        _fire_gather(b, b)

    def _iter(i, _):
        for b in range(NBUF):
            _wait_gather(b)
            _scale(i * NBUF + b, b)
            pltpu.async_copy(fbufs.at[b], acc.at[didx_b.at[b]], ssems.at[b],
                             add=True)
        for b in range(NBUF):
            ck = i * NBUF + b
            pltpu.make_async_copy(fbufs.at[b], acc.at[didx_b.at[b]],
                                  ssems.at[b]).wait()

            @pl.when(ck + NBUF < NCHUNKS)
            def _():
                _fire_gather(ck + NBUF, b)
        return 0
    lax.fori_loop(0, NCHUNKS // NBUF, _iter, 0)

    for ck in range(NBUF * (NCHUNKS // NBUF), NCHUNKS):
        b = ck % NBUF
        _wait_gather(b)
        _scale(ck, b)
        pltpu.sync_copy(fbufs.at[b], acc.at[didx_b.at[b]], add=True)
    plsc.subcore_barrier()

    for t in range(ROWS_PER_TILE // CHUNK):
        r0 = s * ROWS_PER_TILE + t * CHUNK
        pltpu.sync_copy(acc.at[pl.ds(r0, CHUNK)], fbufs.at[0])
        pltpu.sync_copy(fbufs.at[0], out_hbm.at[c, pl.ds(r0, CHUNK)])


_sc_scatter = functools.partial(
    pl.kernel,
    mesh=plsc.VectorSubcoreMesh(core_axis_name="c", subcore_axis_name="s"),
    out_type=jax.ShapeDtypeStruct((NUM_CORES, NPAD, F), jnp.float32),
    compiler_params=pltpu.CompilerParams(use_tc_tiling_on_sc=False),
    scratch_types=[
        pltpu.VMEM((EDGES_PER_TILE,), jnp.int32),
        pltpu.VMEM((EDGES_PER_TILE // 2 + 16,), jnp.int32),
        pltpu.VMEM((NBUF, CHUNK), jnp.int32),
        pltpu.VMEM((NBUF, CHUNK), jnp.int32),
        pltpu.VMEM((NBUF, CHUNK, F // 2), jnp.int32),
        pltpu.VMEM((NBUF, CHUNK, F), jnp.float32),
        pltpu.VMEM_SHARED((NPAD, F), jnp.float32),
        pltpu.SemaphoreType.DMA((NBUF,)),
        pltpu.SemaphoreType.DMA((NBUF,)),
    ],
)(_sc_body)


def kernel(x, edge_index, edge_weight, W_w, b_w, W_lin, b_lin):
    src = edge_index[0].astype(jnp.int32)
    dst = edge_index[1].astype(jnp.int32)
    packed = src | (dst << IDX_BITS)
    wpk = lax.bitcast_convert_type(
        edge_weight.astype(jnp.bfloat16).reshape(E // 2, 2), jnp.int32)

    h = pl.pallas_call(
        _mm1_kernel,
        grid=(N // M_BLK,),
        in_specs=[
            pl.BlockSpec((M_BLK, F), lambda i: (i, 0)),
            pl.BlockSpec((F, F), lambda i: (0, 0)),
            pl.BlockSpec((1, F), lambda i: (0, 0)),
        ],
        out_specs=pl.BlockSpec((M_BLK, F), lambda i: (i, 0)),
        out_shape=jax.ShapeDtypeStruct((N, F), jnp.float32),
    )(x, W_w, b_w.reshape(1, F))

    hp = lax.bitcast_convert_type(
        h.astype(jnp.bfloat16).reshape(N, F // 2, 2), jnp.int32)

    partials = _sc_scatter(hp, packed, wpk)

    out = pl.pallas_call(
        _mm2_kernel,
        grid=(N // M_BLK,),
        in_specs=[
            pl.BlockSpec((NUM_CORES, M_BLK, F), lambda i: (0, i, 0)),
            pl.BlockSpec((F, F), lambda i: (0, 0)),
            pl.BlockSpec((1, F), lambda i: (0, 0)),
        ],
        out_specs=pl.BlockSpec((M_BLK, F), lambda i: (i, 0)),
        out_shape=jax.ShapeDtypeStruct((N, F), jnp.float32),
    )(partials, W_lin[:, _PERM], b_lin.reshape(1, F))
    return out
